# baseline (speedup 1.0000x reference)
import functools

import jax
import jax.numpy as jnp
from jax import lax
from jax.experimental import pallas as pl
from jax.experimental.pallas import tpu as pltpu
from jax.experimental.pallas import tpu_sc as plsc

F = 26
B = 4096
L = 2
V = 100000
D = 32
NF = 13

NC = 2
NS = 16
NW = NC * NS
BPW = B // NW
GF = 4
NG = (F + GF - 1) // GF
CP = NG * GF * D


def _sc_body(tab_hbm, idx_hbm, out_hbm, idxs, gb, pb, gsem, ssem):
    wid = lax.axis_index("s") * NC + lax.axis_index("c")
    b0 = wid * BPW

    pltpu.sync_copy(idx_hbm.at[:, :, pl.ds(b0, BPW)], idxs)

    handles = {}

    def start_gather(f):
        k = f % 2
        if f > 0:
            for l in range(L):
                for c in range(BPW // 16):
                    sl = pl.ds(c * 16, 16)
                    idxs[f, l, sl] = idxs[f, l, sl] + f * V
        for l in range(L):
            handles[(f, l)] = pltpu.async_copy(
                tab_hbm.at[idxs.at[f, l]], gb.at[k, l], gsem.at[k])

    def process(f):
        k = f % 2
        g = f // GF
        kg = g % 2
        handles[(f, 0)].wait()
        handles[(f, 1)].wait()
        if f % GF == 0 and g >= 2:
            handles[("s", g - 2)].wait()
        col = (f % GF) * D

        def body(r, _):
            for c in range(D // 16):
                sl = pl.ds(c * 16, 16)
                pb[kg, r, pl.ds(col + c * 16, 16)] = (
                    gb[k, 0, r, sl] + gb[k, 1, r, sl])
            return 0

        lax.fori_loop(0, BPW, body, 0, unroll=4)

        if f == F - 1 and F % GF != 0:
            z = jnp.zeros((16,), jnp.float32)

            def zbody(r, _):
                for c in range(((GF * D) - (F % GF) * D) // 16):
                    pb[kg, r, pl.ds((F % GF) * D + c * 16, 16)] = z
                return 0

            lax.fori_loop(0, BPW, zbody, 0, unroll=4)

        if f % GF == GF - 1 or f == F - 1:
            handles[("s", g)] = pltpu.async_copy(
                pb.at[kg], out_hbm.at[pl.ds(b0, BPW), pl.ds(g * GF * D, GF * D)],
                ssem.at[kg])

    start_gather(0)
    for f in range(F):
        if f + 1 < F:
            start_gather(f + 1)
        process(f)
    handles[("s", NG - 2)].wait()
    handles[("s", NG - 1)].wait()


def _sc_gather_pool(tab_flat, idx):
    mesh = plsc.VectorSubcoreMesh(
        core_axis_name="c", subcore_axis_name="s", num_cores=NC,
        num_subcores=NS)
    return pl.kernel(
        _sc_body,
        out_type=jax.ShapeDtypeStruct((B, CP), jnp.float32),
        mesh=mesh,
        scratch_types=[
            pltpu.VMEM((F, L, BPW), jnp.int32),
            pltpu.VMEM((2, L, BPW, D), jnp.float32),
            pltpu.VMEM((2, BPW, GF * D), jnp.float32),
            pltpu.SemaphoreType.DMA((2,)),
            pltpu.SemaphoreType.DMA((2,)),
        ],
        compiler_params=pltpu.CompilerParams(use_tc_tiling_on_sc=False),
    )(tab_flat, idx)


BM = 512


def _mlp_body(x_ref, ff_ref, sw1, sb1, sw2, sb2, fw1, fb1, fw2, fb2,
              ow1a, ow1b, ob1, ow2, ob2, ow3, ob3, out_ref):
    mm = functools.partial(jnp.dot, preferred_element_type=jnp.float32)
    s = jax.nn.relu(mm(x_ref[...], sw1[...]) + sb1[...])
    s = jax.nn.relu(mm(s, sw2[...]) + sb2[...])
    f = jax.nn.relu(mm(ff_ref[...], fw1[...]) + fb1[...])
    f = jax.nn.relu(mm(f, fw2[...]) + fb2[...])
    o = jax.nn.relu(mm(s, ow1a[...]) + mm(f, ow1b[...]) + ob1[...])
    o = jax.nn.relu(mm(o, ow2[...]) + ob2[...])
    o = jax.nn.relu(mm(o, ow3[...]) + ob3[...])
    out_ref[...] = o


def _tc_mlp(x, ffp, sw1p, sb1, sw2, sb2, fw1p, fb1, fw2, fb2,
            ow1a, ow1b, ob1, ow2, ob2, ow3, ob3):
    nb = B // BM
    row_spec = lambda c: pl.BlockSpec((BM, c), lambda i: (i, 0))
    full = lambda a: pl.BlockSpec(a.shape, lambda i: (0,) * a.ndim)
    ws = [sw1p, sb1, sw2, sb2, fw1p, fb1, fw2, fb2,
          ow1a, ow1b, ob1, ow2, ob2, ow3, ob3]
    return pl.pallas_call(
        _mlp_body,
        grid=(nb,),
        in_specs=[row_spec(CP), row_spec(128)] + [full(w) for w in ws],
        out_specs=row_spec(1),
        out_shape=jax.ShapeDtypeStruct((B, 1), jnp.float32),
        compiler_params=pltpu.CompilerParams(
            dimension_semantics=("arbitrary",)),
    )(x, ffp, *ws)


def kernel(values, float_features, tables, sw1, sb1, sw2, sb2, fw1, fb1,
           fw2, fb2, ow1, ob1, ow2, ob2, ow3, ob3):
    tab_flat = tables.reshape(F * V, D)
    idx = jnp.transpose(values, (0, 2, 1)).astype(jnp.int32)
    pooled = _sc_gather_pool(tab_flat, idx)

    ffp = jnp.pad(float_features, ((0, 0), (0, 128 - NF)))
    fw1p = jnp.pad(fw1, ((0, 128 - NF), (0, 0)))
    sw1p = jnp.pad(sw1, ((0, CP - F * D), (0, 0)))
    ow1a, ow1b = ow1[:256], ow1[256:]
    out = _tc_mlp(pooled, ffp,
                  sw1p, sb1.reshape(1, -1), sw2, sb2.reshape(1, -1),
                  fw1p, fb1.reshape(1, -1), fw2, fb2.reshape(1, -1),
                  ow1a, ow1b, ob1.reshape(1, -1), ow2, ob2.reshape(1, -1),
                  ow3, ob3.reshape(1, -1))
    return out


# speedup vs baseline: 4.5325x; 4.5325x over previous
import functools

import jax
import jax.numpy as jnp
from jax import lax
from jax.experimental import pallas as pl
from jax.experimental.pallas import tpu as pltpu
from jax.experimental.pallas import tpu_sc as plsc

F = 26
B = 4096
L = 2
V = 100000
D = 32
NF = 13

NC = 2
NS = 16
NW = NC * NS


def _sc_body(tab_hbm, idx_hbm, out_hbm, row_v, idx_v, orow_v, rsem, isem, osem):
    d = lax.axis_index("s") * NC + lax.axis_index("c")

    handles = {}

    def start_row(f):
        handles[("r", f)] = pltpu.async_copy(
            tab_hbm.at[f, d, :], row_v, rsem)

    def start_idx(f):
        handles[("i", f)] = pltpu.async_copy(
            idx_hbm.at[f], idx_v.at[f % 2], isem.at[f % 2])

    def process(f):
        k = f % 2
        handles[("i", f)].wait()
        handles[("r", f)].wait()
        if f >= 2:
            handles[("o", f - 2)].wait()

        def body(i, _):
            sl = pl.ds(i * 16, 16)
            g0 = plsc.load_gather(row_v, [idx_v[k, 0, sl]])
            g1 = plsc.load_gather(row_v, [idx_v[k, 1, sl]])
            orow_v[k, sl] = g0 + g1
            return 0

        lax.fori_loop(0, B // 16, body, 0, unroll=8)
        handles[("o", f)] = pltpu.async_copy(
            orow_v.at[k], out_hbm.at[f * D + d, :], osem.at[k])

    start_idx(0)
    start_row(0)
    start_idx(1)
    for f in range(F):
        process(f)
        if f + 1 < F:
            start_row(f + 1)
        if f + 2 < F:
            start_idx(f + 2)
    handles[("o", F - 2)].wait()
    handles[("o", F - 1)].wait()


def _sc_gather_pool_t(tabT, idx):
    mesh = plsc.VectorSubcoreMesh(
        core_axis_name="c", subcore_axis_name="s", num_cores=NC,
        num_subcores=NS)
    return pl.kernel(
        _sc_body,
        out_type=jax.ShapeDtypeStruct((F * D, B), jnp.float32),
        mesh=mesh,
        scratch_types=[
            pltpu.VMEM((V,), jnp.float32),
            pltpu.VMEM((2, L, B), jnp.int32),
            pltpu.VMEM((2, B), jnp.float32),
            pltpu.SemaphoreType.DMA,
            pltpu.SemaphoreType.DMA((2,)),
            pltpu.SemaphoreType.DMA((2,)),
        ],
        compiler_params=pltpu.CompilerParams(needs_layout_passes=False),
    )(tabT, idx)


BM = 512


def _mlp_body(x_ref, ff_ref, sw1, sb1, sw2, sb2, fw1, fb1, fw2, fb2,
              ow1a, ow1b, ob1, ow2, ob2, ow3, ob3, out_ref):
    mm = functools.partial(jnp.dot, preferred_element_type=jnp.float32)
    s = jax.nn.relu(mm(sw1[...], x_ref[...]) + sb1[...])
    s = jax.nn.relu(mm(sw2[...], s) + sb2[...])
    f = jax.nn.relu(mm(fw1[...], ff_ref[...]) + fb1[...])
    f = jax.nn.relu(mm(fw2[...], f) + fb2[...])
    o = jax.nn.relu(mm(ow1a[...], s) + mm(ow1b[...], f) + ob1[...])
    o = jax.nn.relu(mm(ow2[...], o) + ob2[...])
    o = jax.nn.relu(mm(ow3[...], o) + ob3[...])
    out_ref[...] = o


def _tc_mlp_t(x, ffT, sw1t, sb1, sw2t, sb2, fw1t, fb1, fw2t, fb2,
              ow1at, ow1bt, ob1, ow2t, ob2, ow3t, ob3):
    nb = B // BM
    col_spec = lambda r: pl.BlockSpec((r, BM), lambda i: (0, i))
    full = lambda a: pl.BlockSpec(a.shape, lambda i: (0,) * a.ndim)
    ws = [sw1t, sb1, sw2t, sb2, fw1t, fb1, fw2t, fb2,
          ow1at, ow1bt, ob1, ow2t, ob2, ow3t, ob3]
    return pl.pallas_call(
        _mlp_body,
        grid=(nb,),
        in_specs=[col_spec(F * D), col_spec(NF)] + [full(w) for w in ws],
        out_specs=col_spec(1),
        out_shape=jax.ShapeDtypeStruct((1, B), jnp.float32),
        compiler_params=pltpu.CompilerParams(
            dimension_semantics=("arbitrary",)),
    )(x, ffT, *ws)


def kernel(values, float_features, tables, sw1, sb1, sw2, sb2, fw1, fb1,
           fw2, fb2, ow1, ob1, ow2, ob2, ow3, ob3):
    tabT = jnp.transpose(tables, (0, 2, 1))
    idx = jnp.transpose(values, (0, 2, 1)).astype(jnp.int32)
    s_t = _sc_gather_pool_t(tabT, idx)

    ffT = jnp.transpose(float_features, (1, 0))
    outT = _tc_mlp_t(s_t, ffT,
                     sw1.T, sb1.reshape(-1, 1), sw2.T, sb2.reshape(-1, 1),
                     fw1.T, fb1.reshape(-1, 1), fw2.T, fb2.reshape(-1, 1),
                     ow1[:256].T, ow1[256:].T, ob1.reshape(-1, 1),
                     ow2.T, ob2.reshape(-1, 1), ow3.T, ob3.reshape(-1, 1))
    return outT.T


# speedup vs baseline: 4.5582x; 1.0057x over previous
import functools

import jax
import jax.numpy as jnp
from jax import lax
from jax.experimental import pallas as pl
from jax.experimental.pallas import tpu as pltpu
from jax.experimental.pallas import tpu_sc as plsc

F = 26
B = 4096
L = 2
V = 100000
D = 32
NF = 13

NC = 2
NS = 16
NW = NC * NS


def _sc_body(tab_hbm, idx_hbm, out_hbm, row_v, idx_v, orow_v, rsem, isem, osem):
    d = lax.axis_index("c") * NS + lax.axis_index("s")

    handles = {}

    def start_row(f):
        handles[("r", f)] = pltpu.async_copy(
            tab_hbm.at[f, d, :], row_v, rsem)

    def start_idx(f):
        handles[("i", f)] = pltpu.async_copy(
            idx_hbm.at[f], idx_v.at[f % 2], isem.at[f % 2])

    def process(f):
        k = f % 2
        handles[("i", f)].wait()
        handles[("r", f)].wait()
        if f >= 2:
            handles[("o", f - 2)].wait()

        def body(i, _):
            sl = pl.ds(i * 16, 16)
            g0 = plsc.load_gather(row_v, [idx_v[k, 0, sl]])
            g1 = plsc.load_gather(row_v, [idx_v[k, 1, sl]])
            orow_v[k, sl] = g0 + g1
            return 0

        lax.fori_loop(0, B // 16, body, 0, unroll=8)
        handles[("o", f)] = pltpu.async_copy(
            orow_v.at[k], out_hbm.at[f * D + d, :], osem.at[k])

    start_idx(0)
    start_row(0)
    start_idx(1)
    for f in range(F):
        process(f)
        if f + 1 < F:
            start_row(f + 1)
        if f + 2 < F:
            start_idx(f + 2)
    handles[("o", F - 2)].wait()
    handles[("o", F - 1)].wait()


def _sc_gather_pool_t(tabT, idx):
    mesh = plsc.VectorSubcoreMesh(
        core_axis_name="c", subcore_axis_name="s", num_cores=NC,
        num_subcores=NS)
    return pl.kernel(
        _sc_body,
        out_type=jax.ShapeDtypeStruct((F * D, B), jnp.float32),
        mesh=mesh,
        scratch_types=[
            pltpu.VMEM((V,), jnp.float32),
            pltpu.VMEM((2, L, B), jnp.int32),
            pltpu.VMEM((2, B), jnp.float32),
            pltpu.SemaphoreType.DMA,
            pltpu.SemaphoreType.DMA((2,)),
            pltpu.SemaphoreType.DMA((2,)),
        ],
        compiler_params=pltpu.CompilerParams(needs_layout_passes=False),
    )(tabT, idx)


BM = 512


def _mlp_body(x_ref, ff_ref, sw1, sb1, sw2, sb2, fw1, fb1, fw2, fb2,
              ow1a, ow1b, ob1, ow2, ob2, ow3, ob3, out_ref):
    mm = functools.partial(jnp.dot, preferred_element_type=jnp.float32)
    s = jax.nn.relu(mm(sw1[...], x_ref[...]) + sb1[...])
    s = jax.nn.relu(mm(sw2[...], s) + sb2[...])
    f = jax.nn.relu(mm(fw1[...], ff_ref[...]) + fb1[...])
    f = jax.nn.relu(mm(fw2[...], f) + fb2[...])
    o = jax.nn.relu(mm(ow1a[...], s) + mm(ow1b[...], f) + ob1[...])
    o = jax.nn.relu(mm(ow2[...], o) + ob2[...])
    o = jax.nn.relu(mm(ow3[...], o) + ob3[...])
    out_ref[...] = o


def _tc_mlp_t(x, ffT, sw1t, sb1, sw2t, sb2, fw1t, fb1, fw2t, fb2,
              ow1at, ow1bt, ob1, ow2t, ob2, ow3t, ob3):
    nb = B // BM
    col_spec = lambda r: pl.BlockSpec((r, BM), lambda i: (0, i))
    full = lambda a: pl.BlockSpec(a.shape, lambda i: (0,) * a.ndim)
    ws = [sw1t, sb1, sw2t, sb2, fw1t, fb1, fw2t, fb2,
          ow1at, ow1bt, ob1, ow2t, ob2, ow3t, ob3]
    return pl.pallas_call(
        _mlp_body,
        grid=(nb,),
        in_specs=[col_spec(F * D), col_spec(NF)] + [full(w) for w in ws],
        out_specs=col_spec(1),
        out_shape=jax.ShapeDtypeStruct((1, B), jnp.float32),
        compiler_params=pltpu.CompilerParams(
            dimension_semantics=("arbitrary",)),
    )(x, ffT, *ws)


def kernel(values, float_features, tables, sw1, sb1, sw2, sb2, fw1, fb1,
           fw2, fb2, ow1, ob1, ow2, ob2, ow3, ob3):
    tabT = jnp.transpose(tables, (0, 2, 1))
    idx = jnp.transpose(values, (0, 2, 1)).astype(jnp.int32)
    s_t = _sc_gather_pool_t(tabT, idx)

    ffT = jnp.transpose(float_features, (1, 0))
    outT = _tc_mlp_t(s_t, ffT,
                     sw1.T, sb1.reshape(-1, 1), sw2.T, sb2.reshape(-1, 1),
                     fw1.T, fb1.reshape(-1, 1), fw2.T, fb2.reshape(-1, 1),
                     ow1[:256].T, ow1[256:].T, ob1.reshape(-1, 1),
                     ow2.T, ob2.reshape(-1, 1), ow3.T, ob3.reshape(-1, 1))
    return outT.T


# speedup vs baseline: 4.8873x; 1.0722x over previous
import functools

import jax
import jax.numpy as jnp
from jax import lax
from jax.experimental import pallas as pl
from jax.experimental.pallas import tpu as pltpu
from jax.experimental.pallas import tpu_sc as plsc

F = 26
B = 4096
L = 2
V = 100000
D = 32
NF = 13

NC = 2
NS = 16
NW = NC * NS


def _sc_body(tab_hbm, idx_hbm, out_hbm, row_v, idx_v, orow_v, rsem, isem, osem):
    d = lax.axis_index("c") * NS + lax.axis_index("s")
    off = lax.rem(d, F)

    handles = {}

    def fld(f):
        return lax.rem(f + off, F)

    def start_row(f):
        handles[("r", f)] = pltpu.async_copy(
            tab_hbm.at[fld(f), d, :], row_v, rsem)

    def start_idx(f):
        handles[("i", f)] = pltpu.async_copy(
            idx_hbm.at[fld(f)], idx_v.at[f % 2], isem.at[f % 2])

    def process(f):
        k = f % 2
        handles[("i", f)].wait()
        handles[("r", f)].wait()
        if f >= 2:
            handles[("o", f - 2)].wait()

        def body(i, _):
            sl = pl.ds(i * 16, 16)
            g0 = plsc.load_gather(row_v, [idx_v[k, 0, sl]])
            g1 = plsc.load_gather(row_v, [idx_v[k, 1, sl]])
            orow_v[k, sl] = g0 + g1
            return 0

        lax.fori_loop(0, B // 16, body, 0, unroll=8)
        handles[("o", f)] = pltpu.async_copy(
            orow_v.at[k], out_hbm.at[fld(f) * D + d, :], osem.at[k])

    start_idx(0)
    start_row(0)
    start_idx(1)
    for f in range(F):
        process(f)
        if f + 1 < F:
            start_row(f + 1)
        if f + 2 < F:
            start_idx(f + 2)
    handles[("o", F - 2)].wait()
    handles[("o", F - 1)].wait()


def _sc_gather_pool_t(tabT, idx):
    mesh = plsc.VectorSubcoreMesh(
        core_axis_name="c", subcore_axis_name="s", num_cores=NC,
        num_subcores=NS)
    return pl.kernel(
        _sc_body,
        out_type=jax.ShapeDtypeStruct((F * D, B), jnp.float32),
        mesh=mesh,
        scratch_types=[
            pltpu.VMEM((V,), jnp.float32),
            pltpu.VMEM((2, L, B), jnp.int32),
            pltpu.VMEM((2, B), jnp.float32),
            pltpu.SemaphoreType.DMA,
            pltpu.SemaphoreType.DMA((2,)),
            pltpu.SemaphoreType.DMA((2,)),
        ],
        compiler_params=pltpu.CompilerParams(needs_layout_passes=False),
    )(tabT, idx)


BM = 512


def _mlp_body(x_ref, ff_ref, sw1, sb1, sw2, sb2, fw1, fb1, fw2, fb2,
              ow1a, ow1b, ob1, ow2, ob2, ow3, ob3, out_ref):
    mm = functools.partial(jnp.dot, preferred_element_type=jnp.float32)
    s = jax.nn.relu(mm(sw1[...], x_ref[...]) + sb1[...])
    s = jax.nn.relu(mm(sw2[...], s) + sb2[...])
    f = jax.nn.relu(mm(fw1[...], ff_ref[...]) + fb1[...])
    f = jax.nn.relu(mm(fw2[...], f) + fb2[...])
    o = jax.nn.relu(mm(ow1a[...], s) + mm(ow1b[...], f) + ob1[...])
    o = jax.nn.relu(mm(ow2[...], o) + ob2[...])
    o = jax.nn.relu(mm(ow3[...], o) + ob3[...])
    out_ref[...] = o


def _tc_mlp_t(x, ffT, sw1t, sb1, sw2t, sb2, fw1t, fb1, fw2t, fb2,
              ow1at, ow1bt, ob1, ow2t, ob2, ow3t, ob3):
    nb = B // BM
    col_spec = lambda r: pl.BlockSpec((r, BM), lambda i: (0, i))
    full = lambda a: pl.BlockSpec(a.shape, lambda i: (0,) * a.ndim)
    ws = [sw1t, sb1, sw2t, sb2, fw1t, fb1, fw2t, fb2,
          ow1at, ow1bt, ob1, ow2t, ob2, ow3t, ob3]
    return pl.pallas_call(
        _mlp_body,
        grid=(nb,),
        in_specs=[col_spec(F * D), col_spec(NF)] + [full(w) for w in ws],
        out_specs=col_spec(1),
        out_shape=jax.ShapeDtypeStruct((1, B), jnp.float32),
        compiler_params=pltpu.CompilerParams(
            dimension_semantics=("arbitrary",)),
    )(x, ffT, *ws)


def kernel(values, float_features, tables, sw1, sb1, sw2, sb2, fw1, fb1,
           fw2, fb2, ow1, ob1, ow2, ob2, ow3, ob3):
    tabT = jnp.transpose(tables, (0, 2, 1))
    idx = jnp.transpose(values, (0, 2, 1)).astype(jnp.int32)
    s_t = _sc_gather_pool_t(tabT, idx)

    ffT = jnp.transpose(float_features, (1, 0))
    outT = _tc_mlp_t(s_t, ffT,
                     sw1.T, sb1.reshape(-1, 1), sw2.T, sb2.reshape(-1, 1),
                     fw1.T, fb1.reshape(-1, 1), fw2.T, fb2.reshape(-1, 1),
                     ow1[:256].T, ow1[256:].T, ob1.reshape(-1, 1),
                     ow2.T, ob2.reshape(-1, 1), ow3.T, ob3.reshape(-1, 1))
    return outT.T


# speedup vs baseline: 5.2429x; 1.0728x over previous
import functools

import jax
import jax.numpy as jnp
from jax import lax
from jax.experimental import pallas as pl
from jax.experimental.pallas import tpu as pltpu
from jax.experimental.pallas import tpu_sc as plsc

F = 26
B = 4096
L = 2
V = 100000
D = 32
NF = 13

NC = 2
NS = 16
NW = NC * NS


VSPLIT = 49920
VVHI = V - VSPLIT


def _sc_body(tab_hbm, idx_hbm, out_hbm, lo_v, hi_v, idx_v, orow_v,
             rsem, isem, osem):
    d = lax.axis_index("c") * NS + lax.axis_index("s")
    off = lax.rem(d, F)

    handles = {}

    def fld(f):
        return lax.rem(f + off, F)

    def start_lo(f):
        handles[("a", f)] = pltpu.async_copy(
            tab_hbm.at[fld(f), d, pl.ds(0, VSPLIT)], lo_v, rsem.at[0])

    def start_hi(f):
        handles[("b", f)] = pltpu.async_copy(
            tab_hbm.at[fld(f), d, pl.ds(VSPLIT, VVHI)], hi_v, rsem.at[1])

    def start_idx(f):
        handles[("i", f)] = pltpu.async_copy(
            idx_hbm.at[fld(f)], idx_v.at[f % 2], isem.at[f % 2])

    def extract(f, h):
        k = f % 2
        buf = lo_v if h == 0 else hi_v

        def body(i, _):
            sl = pl.ds(i * 16, 16)
            acc = orow_v[k, sl] if h == 1 else None
            for l in range(L):
                v = idx_v[k, l, sl]
                if h == 0:
                    m = v < VSPLIT
                    vloc = jnp.minimum(v, VSPLIT - 1)
                else:
                    m = v >= VSPLIT
                    vloc = jnp.maximum(v - VSPLIT, 0)
                g = jnp.where(m, plsc.load_gather(buf, [vloc], mask=m), 0.0)
                acc = g if acc is None else acc + g
            orow_v[k, sl] = acc
            return 0

        lax.fori_loop(0, B // 16, body, 0, unroll=4)

    start_idx(0)
    start_lo(0)
    start_hi(0)
    start_idx(1)
    for f in range(F):
        k = f % 2
        handles[("i", f)].wait()
        handles[("a", f)].wait()
        if f >= 2:
            handles[("o", f - 2)].wait()
        extract(f, 0)
        if f + 1 < F:
            start_lo(f + 1)
        handles[("b", f)].wait()
        extract(f, 1)
        if f + 1 < F:
            start_hi(f + 1)
        if f + 2 < F:
            start_idx(f + 2)
        handles[("o", f)] = pltpu.async_copy(
            orow_v.at[k], out_hbm.at[fld(f) * D + d, :], osem.at[k])
    handles[("o", F - 2)].wait()
    handles[("o", F - 1)].wait()


def _sc_gather_pool_t(tabT, idx):
    mesh = plsc.VectorSubcoreMesh(
        core_axis_name="c", subcore_axis_name="s", num_cores=NC,
        num_subcores=NS)
    return pl.kernel(
        _sc_body,
        out_type=jax.ShapeDtypeStruct((F * D, B), jnp.float32),
        mesh=mesh,
        scratch_types=[
            pltpu.VMEM((VSPLIT,), jnp.float32),
            pltpu.VMEM((VVHI,), jnp.float32),
            pltpu.VMEM((2, L, B), jnp.int32),
            pltpu.VMEM((2, B), jnp.float32),
            pltpu.SemaphoreType.DMA((2,)),
            pltpu.SemaphoreType.DMA((2,)),
            pltpu.SemaphoreType.DMA((2,)),
        ],
        compiler_params=pltpu.CompilerParams(needs_layout_passes=False),
    )(tabT, idx)


BM = 512


def _mlp_body(x_ref, ff_ref, sw1, sb1, sw2, sb2, fw1, fb1, fw2, fb2,
              ow1a, ow1b, ob1, ow2, ob2, ow3, ob3, out_ref):
    mm = functools.partial(jnp.dot, preferred_element_type=jnp.float32)
    s = jax.nn.relu(mm(sw1[...], x_ref[...]) + sb1[...])
    s = jax.nn.relu(mm(sw2[...], s) + sb2[...])
    f = jax.nn.relu(mm(fw1[...], ff_ref[...]) + fb1[...])
    f = jax.nn.relu(mm(fw2[...], f) + fb2[...])
    o = jax.nn.relu(mm(ow1a[...], s) + mm(ow1b[...], f) + ob1[...])
    o = jax.nn.relu(mm(ow2[...], o) + ob2[...])
    o = jax.nn.relu(mm(ow3[...], o) + ob3[...])
    out_ref[...] = o


def _tc_mlp_t(x, ffT, sw1t, sb1, sw2t, sb2, fw1t, fb1, fw2t, fb2,
              ow1at, ow1bt, ob1, ow2t, ob2, ow3t, ob3):
    nb = B // BM
    col_spec = lambda r: pl.BlockSpec((r, BM), lambda i: (0, i))
    full = lambda a: pl.BlockSpec(a.shape, lambda i: (0,) * a.ndim)
    ws = [sw1t, sb1, sw2t, sb2, fw1t, fb1, fw2t, fb2,
          ow1at, ow1bt, ob1, ow2t, ob2, ow3t, ob3]
    return pl.pallas_call(
        _mlp_body,
        grid=(nb,),
        in_specs=[col_spec(F * D), col_spec(NF)] + [full(w) for w in ws],
        out_specs=col_spec(1),
        out_shape=jax.ShapeDtypeStruct((1, B), jnp.float32),
        compiler_params=pltpu.CompilerParams(
            dimension_semantics=("arbitrary",)),
    )(x, ffT, *ws)


def kernel(values, float_features, tables, sw1, sb1, sw2, sb2, fw1, fb1,
           fw2, fb2, ow1, ob1, ow2, ob2, ow3, ob3):
    tabT = jnp.transpose(tables, (0, 2, 1))
    idx = jnp.transpose(values, (0, 2, 1)).astype(jnp.int32)
    s_t = _sc_gather_pool_t(tabT, idx)

    ffT = jnp.transpose(float_features, (1, 0))
    outT = _tc_mlp_t(s_t, ffT,
                     sw1.T, sb1.reshape(-1, 1), sw2.T, sb2.reshape(-1, 1),
                     fw1.T, fb1.reshape(-1, 1), fw2.T, fb2.reshape(-1, 1),
                     ow1[:256].T, ow1[256:].T, ob1.reshape(-1, 1),
                     ow2.T, ob2.reshape(-1, 1), ow3.T, ob3.reshape(-1, 1))
    return outT.T
